# Initial kernel scaffold; baseline (speedup 1.0000x reference)
import jax
import jax.numpy as jnp
from jax.experimental import pallas as pl


def kernel(feat):
    raise NotImplementedError("write your pallas kernel here")


# speedup vs baseline: 674.9090x; 674.9090x over previous
import numpy as np
import jax
import jax.numpy as jnp
from jax.experimental import pallas as pl
from jax.experimental.pallas import tpu as pltpu

_NUMANGLE = 100
_NUMRHO = 100


def _rho_table(H, W, numangle, numrho):
    irho = float(int(np.sqrt(H * H + W * W) + 1)) / float(numrho - 1)
    itheta = np.pi / numangle
    angles = jnp.arange(numangle, dtype=jnp.float32) * itheta
    tabCos = jnp.cos(angles) / irho
    tabSin = jnp.sin(angles) / irho
    xs = jnp.arange(W, dtype=jnp.float32) - (W // 2)
    ys = jnp.arange(H, dtype=jnp.float32) - (H // 2)
    r = jnp.round(xs[None, None, :] * tabCos[:, None, None]
                  + ys[None, :, None] * tabSin[:, None, None]).astype(jnp.int32)
    r = r + numrho // 2
    r = jnp.clip(r, 0, numrho - 1)
    return r.reshape(numangle, 1, H * W)


def _dht_body(r_ref, ft_ref, out_ref):
    P = ft_ref.shape[0]
    r = r_ref[0]
    rho = jax.lax.broadcasted_iota(jnp.int32, (_NUMRHO, P), 0)
    s = jnp.where(jnp.broadcast_to(r, (_NUMRHO, P)) == rho,
                  jnp.float32(1.0), jnp.float32(0.0))
    out_ref[0] = jnp.dot(s, ft_ref[...], preferred_element_type=jnp.float32)


def kernel(feat):
    N, C, H, W = feat.shape
    NC, P = N * C, H * W
    r = _rho_table(H, W, _NUMANGLE, _NUMRHO)
    ft = feat.reshape(NC, P).T

    out = pl.pallas_call(
        _dht_body,
        grid=(_NUMANGLE,),
        in_specs=[
            pl.BlockSpec((1, 1, P), lambda a: (a, 0, 0)),
            pl.BlockSpec((P, NC), lambda a: (0, 0)),
        ],
        out_specs=pl.BlockSpec((1, _NUMRHO, NC), lambda a: (a, 0, 0)),
        out_shape=jax.ShapeDtypeStruct((_NUMANGLE, _NUMRHO, NC), jnp.float32),
    )(r, ft)

    return out.transpose(2, 0, 1).reshape(N, C, _NUMANGLE, _NUMRHO)
